# baseline (speedup 1.0000x reference)
import jax
import jax.numpy as jnp
from jax import lax
from jax.experimental import pallas as pl
from jax.experimental.pallas import tpu as pltpu
from jax.experimental.pallas import tpu_sc as plsc

R = 32
N = 1_000_000
STEP = 131_072
NSUB = STEP // 2_048
FB = STEP // NSUB
NB = 512
LASTB = (N + FB - 1) // FB - 1
TCSTEPS = (N + STEP - 1) // STEP
TPB = FB // 128
NTILES = (N + 127) // 128
LANES = 16
DMA_ROUND = 16


def _tree_sum(vs):
    vs = list(vs)
    while len(vs) > 1:
        nxt = [a + b for a, b in zip(vs[::2], vs[1::2])]
        if len(vs) % 2:
            nxt.append(vs[-1])
        vs = nxt
    return vs[0]


def _tc_body(x_ref, o_ref):
    b = pl.program_id(0)
    lane = lax.broadcasted_iota(jnp.int32, (R, NB), 1)

    @pl.when(b == 0)
    def _init():
        o_ref[...] = jnp.zeros((R, NB), jnp.float32)

    @pl.when(b < TCSTEPS - 1)
    def _full():
        acc = jnp.zeros((R, NB), jnp.float32)
        for q in range(NSUB):
            sums = jnp.sum(x_ref[:, q * FB:(q + 1) * FB], axis=1,
                           keepdims=True)
            acc = acc + jnp.where(lane == b * NSUB + q, sums,
                                  jnp.float32(0.0))
        o_ref[...] = o_ref[...] + acc

    @pl.when(b == TCSTEPS - 1)
    def _tail():
        bb = TCSTEPS - 1
        acc = jnp.zeros((R, NB), jnp.float32)
        for q in range(NSUB):
            colq0 = bb * STEP + q * FB
            if colq0 >= N:
                break
            xq = x_ref[:, q * FB:(q + 1) * FB]
            if colq0 + FB > N:
                cols = colq0 + lax.broadcasted_iota(jnp.int32, (R, FB), 1)
                xq = jnp.where(cols < N, xq, jnp.float32(0.0))
            sums = jnp.sum(xq, axis=1, keepdims=True)
            acc = acc + jnp.where(lane == bb * NSUB + q, sums,
                                  jnp.float32(0.0))
        o_ref[...] = o_ref[...] + acc


def _block_sums(x):
    return pl.pallas_call(
        _tc_body,
        grid=(TCSTEPS,),
        in_specs=[pl.BlockSpec(
            (R, STEP), lambda b: (0, jnp.minimum(b, TCSTEPS - 1)))],
        out_specs=pl.BlockSpec((R, NB), lambda b: (0, 0)),
        out_shape=jax.ShapeDtypeStruct((R, NB), jnp.float32),
    )(x)


def _sc_body(x_hbm, b_hbm, u_hbm, out_hbm, bv, tbuf, u_v, flag_v, semf):
    wid = lax.axis_index("s") * 2 + lax.axis_index("c")
    rr = wid % 8
    rg8 = pl.multiple_of(wid - rr, 8)
    pltpu.sync_copy(u_hbm.at[pl.ds(wid * LANES, LANES)], u_v)
    pltpu.sync_copy(b_hbm.at[pl.ds(wid * NB, NB)], bv)

    vregs = [bv[pl.ds(i * LANES, LANES)] for i in range(NB // LANES)]
    total = jnp.sum(_tree_sum(vregs))
    u_s = u_v[...][0]
    t = u_s * total
    tv = jnp.full((LANES,), t)

    run = jnp.float32(0.0)
    nbv = jnp.zeros((LANES,), jnp.int32)
    pv = jnp.zeros((LANES,), jnp.float32)
    for i in range(NB // LANES):
        v = vregs[i]
        c = plsc.cumsum(v) + jnp.full((LANES,), run)
        m = c < tv
        nbv = nbv + m.astype(jnp.int32)
        pv = pv + jnp.where(m, v, jnp.float32(0.0))
        run = run + jnp.sum(v)
    b_star = jnp.minimum(jnp.sum(nbv), LASTB)
    prefix = jnp.sum(pv)

    base_tc = b_star * TPB
    iota = lax.iota(jnp.int32, LANES)
    for k0 in range(0, TPB, DMA_ROUND):
        hs = []
        for k in range(k0, k0 + DMA_ROUND):
            tc = jnp.minimum(base_tc + k, NTILES - 1)
            cb = pl.multiple_of(tc * 128, 128)
            hs.append(pltpu.async_copy(
                x_hbm.at[pl.ds(rg8, 8), pl.ds(cb, 128)], tbuf.at[k], semf))
        for h in hs:
            h.wait()

    tile_sums = []
    for k in range(TPB):
        real = base_tc + k < NTILES
        colbase = jnp.minimum(base_tc + k, NTILES - 1) * 128
        parts = []
        for j in range(8):
            v = tbuf[k, rr, pl.ds(j * LANES, LANES)]
            valid = ((colbase + j * LANES + iota) < N) & jnp.full(
                (LANES,), real)
            parts.append(jnp.where(valid, v, jnp.float32(0.0)))
        tile_sums.append(jnp.sum(_tree_sum(parts)))

    run2 = prefix
    ntile = jnp.int32(0)
    pfx2 = prefix
    for s in tile_sums:
        run2 = run2 + s
        below = run2 < t
        ntile = ntile + below.astype(jnp.int32)
        pfx2 = pfx2 + jnp.where(below, s, jnp.float32(0.0))
    k_star = jnp.minimum(ntile, TPB - 1)

    kcol = jnp.minimum(base_tc + k_star, NTILES - 1) * 128
    kreal = jnp.full((LANES,), base_tc + k_star < NTILES)
    run3 = pfx2
    cnt1 = jnp.zeros((LANES,), jnp.int32)
    cnt2 = jnp.zeros((LANES,), jnp.int32)
    for j in range(8):
        v = tbuf[k_star, rr, pl.ds(j * LANES, LANES)]
        valid = ((kcol + j * LANES + iota) < N) & kreal
        vm = jnp.where(valid, v, jnp.float32(0.0))
        absc = plsc.cumsum(vm) + jnp.full((LANES,), run3)
        m = (absc < tv) & valid
        cnt1 = cnt1 + m.astype(jnp.int32)
        cnt2 = cnt2 + m.astype(jnp.int32)
        run3 = run3 + jnp.sum(vm)

    idx1 = b_star * FB + k_star * 128 + jnp.sum(cnt1)
    idx2 = b_star * FB + k_star * 128 + jnp.sum(cnt2)
    neq = idx1 != idx2
    flag_v[...] = jnp.full((LANES,), jnp.where(neq, 1.0, 0.0)
                           .astype(jnp.float32))
    pltpu.sync_copy(flag_v, out_hbm.at[pl.ds(wid * LANES, LANES)])


def kernel(x):
    u = jax.random.uniform(jax.random.key(42), (R, 1), dtype=jnp.float32)
    ub = jnp.broadcast_to(u, (R, LANES)).reshape(R * LANES)
    bsum = _block_sums(x).reshape(R * NB)
    mesh = plsc.VectorSubcoreMesh(core_axis_name="c", subcore_axis_name="s",
                                  num_cores=2, num_subcores=16)
    run = pl.kernel(
        _sc_body,
        out_type=jax.ShapeDtypeStruct((R * LANES,), jnp.float32),
        mesh=mesh,
        scratch_types=[
            pltpu.VMEM((NB,), jnp.float32),
            pltpu.VMEM((TPB, 8, 128), jnp.float32),
            pltpu.VMEM((LANES,), jnp.float32),
            pltpu.VMEM((LANES,), jnp.float32),
            pltpu.SemaphoreType.DMA,
        ],
        compiler_params=pltpu.CompilerParams(needs_layout_passes=False),
    )
    flags = run(x, bsum, ub)
    return jnp.any(flags != 0.0).astype(jnp.float32)


# speedup vs baseline: 1.0047x; 1.0047x over previous
import jax
import jax.numpy as jnp
from jax import lax
from jax.experimental import pallas as pl
from jax.experimental.pallas import tpu as pltpu
from jax.experimental.pallas import tpu_sc as plsc

R = 32
N = 1_000_000
STEP = 131_072
NSUB = STEP // 2_048
FB = STEP // NSUB
NB = 512
LASTB = (N + FB - 1) // FB - 1
TCSTEPS = (N + STEP - 1) // STEP
TPB = FB // 128
NTILES = (N + 127) // 128
LANES = 16
DMA_ROUND = 16


def _tree_sum(vs):
    vs = list(vs)
    while len(vs) > 1:
        nxt = [a + b for a, b in zip(vs[::2], vs[1::2])]
        if len(vs) % 2:
            nxt.append(vs[-1])
        vs = nxt
    return vs[0]


def _tc_body(x_ref, o_ref):
    b = pl.program_id(0)
    lane = lax.broadcasted_iota(jnp.int32, (R, NB), 1)

    @pl.when(b == 0)
    def _init():
        o_ref[...] = jnp.zeros((R, NB), jnp.float32)

    @pl.when(b < TCSTEPS - 1)
    def _full():
        acc = jnp.zeros((R, NB), jnp.float32)
        for q in range(NSUB):
            sums = jnp.sum(x_ref[:, q * FB:(q + 1) * FB], axis=1,
                           keepdims=True)
            acc = acc + jnp.where(lane == b * NSUB + q, sums,
                                  jnp.float32(0.0))
        o_ref[...] = o_ref[...] + acc

    @pl.when(b == TCSTEPS - 1)
    def _tail():
        bb = TCSTEPS - 1
        acc = jnp.zeros((R, NB), jnp.float32)
        for q in range(NSUB):
            colq0 = bb * STEP + q * FB
            if colq0 >= N:
                break
            xq = x_ref[:, q * FB:(q + 1) * FB]
            if colq0 + FB > N:
                cols = colq0 + lax.broadcasted_iota(jnp.int32, (R, FB), 1)
                xq = jnp.where(cols < N, xq, jnp.float32(0.0))
            sums = jnp.sum(xq, axis=1, keepdims=True)
            acc = acc + jnp.where(lane == bb * NSUB + q, sums,
                                  jnp.float32(0.0))
        o_ref[...] = o_ref[...] + acc


def _block_sums(x):
    return pl.pallas_call(
        _tc_body,
        grid=(TCSTEPS,),
        in_specs=[pl.BlockSpec(
            (R, STEP), lambda b: (0, jnp.minimum(b, TCSTEPS - 1)))],
        out_specs=pl.BlockSpec((R, NB), lambda b: (0, 0)),
        out_shape=jax.ShapeDtypeStruct((R, NB), jnp.float32),
    )(x)


def _sc_body(x_hbm, b_hbm, u_hbm, out_hbm, bv, tbuf, u_v, flag_v, semf):
    wid = lax.axis_index("s") * 2 + lax.axis_index("c")
    rr = wid % 8
    rg8 = pl.multiple_of(wid - rr, 8)
    pltpu.sync_copy(u_hbm.at[pl.ds(wid * LANES, LANES)], u_v)
    pltpu.sync_copy(b_hbm.at[pl.ds(wid * NB, NB)], bv)

    vregs = [bv[pl.ds(i * LANES, LANES)] for i in range(NB // LANES)]
    total = jnp.sum(_tree_sum(vregs))
    u_s = u_v[...][0]
    t = u_s * total
    tv = jnp.full((LANES,), t)

    run = jnp.float32(0.0)
    nbv = jnp.zeros((LANES,), jnp.int32)
    pv = jnp.zeros((LANES,), jnp.float32)
    for i in range(NB // LANES):
        v = vregs[i]
        c = plsc.cumsum(v) + jnp.full((LANES,), run)
        m = c < tv
        nbv = nbv + m.astype(jnp.int32)
        pv = pv + jnp.where(m, v, jnp.float32(0.0))
        run = run + jnp.sum(v)
    b_star = jnp.minimum(jnp.sum(nbv), LASTB)
    prefix = jnp.sum(pv)

    base_tc = b_star * TPB
    iota = lax.iota(jnp.int32, LANES)
    tc0 = jnp.minimum(base_tc, NTILES - TPB)
    shift = base_tc - tc0
    cb = pl.multiple_of(tc0 * 128, 128)
    pltpu.async_copy(
        x_hbm.at[pl.ds(rg8, 8), pl.ds(cb, TPB * 128)], tbuf, semf).wait()

    tile_sums = []
    for k in range(TPB):
        koff = jnp.minimum(k + shift, TPB - 1)
        colbase = (base_tc + k) * 128
        parts = []
        for j in range(8):
            v = tbuf[rr, pl.ds(koff * 128 + j * LANES, LANES)]
            valid = (colbase + j * LANES + iota) < N
            parts.append(jnp.where(valid, v, jnp.float32(0.0)))
        tile_sums.append(jnp.sum(_tree_sum(parts)))

    run2 = prefix
    ntile = jnp.int32(0)
    pfx2 = prefix
    for s in tile_sums:
        run2 = run2 + s
        below = run2 < t
        ntile = ntile + below.astype(jnp.int32)
        pfx2 = pfx2 + jnp.where(below, s, jnp.float32(0.0))
    k_star = jnp.minimum(ntile, TPB - 1)

    kshift = jnp.minimum(k_star + shift, TPB - 1)
    kcol = (base_tc + k_star) * 128
    run3 = pfx2
    cnt1 = jnp.zeros((LANES,), jnp.int32)
    cnt2 = jnp.zeros((LANES,), jnp.int32)
    for j in range(8):
        v = tbuf[rr, pl.ds(kshift * 128 + j * LANES, LANES)]
        valid = (kcol + j * LANES + iota) < N
        vm = jnp.where(valid, v, jnp.float32(0.0))
        absc = plsc.cumsum(vm) + jnp.full((LANES,), run3)
        m = (absc < tv) & valid
        cnt1 = cnt1 + m.astype(jnp.int32)
        cnt2 = cnt2 + m.astype(jnp.int32)
        run3 = run3 + jnp.sum(vm)

    idx1 = b_star * FB + k_star * 128 + jnp.sum(cnt1)
    idx2 = b_star * FB + k_star * 128 + jnp.sum(cnt2)
    neq = idx1 != idx2
    flag_v[...] = jnp.full((LANES,), jnp.where(neq, 1.0, 0.0)
                           .astype(jnp.float32))
    pltpu.sync_copy(flag_v, out_hbm.at[pl.ds(wid * LANES, LANES)])


def kernel(x):
    u = jax.random.uniform(jax.random.key(42), (R, 1), dtype=jnp.float32)
    ub = jnp.broadcast_to(u, (R, LANES)).reshape(R * LANES)
    bsum = _block_sums(x).reshape(R * NB)
    mesh = plsc.VectorSubcoreMesh(core_axis_name="c", subcore_axis_name="s",
                                  num_cores=2, num_subcores=16)
    run = pl.kernel(
        _sc_body,
        out_type=jax.ShapeDtypeStruct((R * LANES,), jnp.float32),
        mesh=mesh,
        scratch_types=[
            pltpu.VMEM((NB,), jnp.float32),
            pltpu.VMEM((8, TPB * 128), jnp.float32),
            pltpu.VMEM((LANES,), jnp.float32),
            pltpu.VMEM((LANES,), jnp.float32),
            pltpu.SemaphoreType.DMA,
        ],
        compiler_params=pltpu.CompilerParams(needs_layout_passes=False),
    )
    flags = run(x, bsum, ub)
    return jnp.any(flags != 0.0).astype(jnp.float32)
